# baseline (speedup 1.0000x reference)
import functools

import jax
import jax.numpy as jnp
from jax.experimental import pallas as pl
from jax.experimental.pallas import tpu as pltpu

_BLOCK = 4096


def _prep_kernel(sim_ref, gates_ref, simn_ref, thr_ref):
    sim = sim_ref[...]
    col_n = jnp.sqrt(jnp.sum(sim * sim, axis=0, keepdims=True))
    simn_ref[...] = sim / jnp.maximum(col_n, 1e-12)
    thr_ref[...] = jax.nn.sigmoid(gates_ref[...])


def _gate_kernel(x_ref, simn_ref, thr_ref, probs_ref, pre_ref, mask_ref):
    x = x_ref[...]
    raw = jnp.dot(x, simn_ref[...], preferred_element_type=jnp.float32)
    row_n = jnp.sqrt(jnp.sum(x * x, axis=1, keepdims=True))
    logits = raw / jnp.maximum(row_n, 1e-12)

    pre = logits - thr_ref[...]
    gated = jnp.maximum(pre, 0.0)
    active = gated > 0.0

    neg = -jnp.finfo(jnp.float32).max
    masked = jnp.where(active, gated, neg)
    m = jnp.max(masked, axis=1, keepdims=True)
    e = jnp.exp(masked - m)
    probs = e / jnp.sum(e, axis=1, keepdims=True)

    probs_ref[...] = probs
    pre_ref[...] = pre
    mask_ref[...] = active.astype(jnp.float32)


@functools.partial(jax.jit)
def kernel(x, sim_matrix, gates):
    n_tokens, hidden = x.shape
    n_experts = sim_matrix.shape[1]
    gates2d = gates.reshape(1, n_experts)

    sim_n, thr = pl.pallas_call(
        _prep_kernel,
        in_specs=[
            pl.BlockSpec((hidden, n_experts), lambda: (0, 0)),
            pl.BlockSpec((1, n_experts), lambda: (0, 0)),
        ],
        out_specs=[
            pl.BlockSpec((hidden, n_experts), lambda: (0, 0)),
            pl.BlockSpec((1, n_experts), lambda: (0, 0)),
        ],
        out_shape=[
            jax.ShapeDtypeStruct((hidden, n_experts), jnp.float32),
            jax.ShapeDtypeStruct((1, n_experts), jnp.float32),
        ],
    )(sim_matrix, gates2d)

    grid = (n_tokens // _BLOCK,)
    out_shape = jax.ShapeDtypeStruct((n_tokens, n_experts), jnp.float32)
    out_spec = pl.BlockSpec((_BLOCK, n_experts), lambda i: (i, 0))

    probs, pre, mask = pl.pallas_call(
        _gate_kernel,
        grid=grid,
        in_specs=[
            pl.BlockSpec((_BLOCK, hidden), lambda i: (i, 0)),
            pl.BlockSpec((hidden, n_experts), lambda i: (0, 0)),
            pl.BlockSpec((1, n_experts), lambda i: (0, 0)),
        ],
        out_specs=[out_spec, out_spec, out_spec],
        out_shape=[out_shape, out_shape, out_shape],
        compiler_params=pltpu.CompilerParams(
            dimension_semantics=("arbitrary",),
        ),
    )(x, sim_n, thr)

    return probs, pre, mask


# speedup vs baseline: 1.0236x; 1.0236x over previous
import functools

import jax
import jax.numpy as jnp
from jax.experimental import pallas as pl
from jax.experimental.pallas import tpu as pltpu

_BLOCK = 4096


def _gate_kernel(x_ref, sim_ref, gates_ref,
                 probs_ref, pre_ref, mask_ref,
                 simn_ref, thr_ref):
    @pl.when(pl.program_id(0) == 0)
    def _prep():
        sim = sim_ref[...]
        col_n = jnp.sqrt(jnp.sum(sim * sim, axis=0, keepdims=True))
        simn_ref[...] = sim / jnp.maximum(col_n, 1e-12)
        thr_ref[...] = jax.nn.sigmoid(gates_ref[...])

    x = x_ref[...]
    raw = jnp.dot(x, simn_ref[...], preferred_element_type=jnp.float32)
    row_n = jnp.sqrt(jnp.sum(x * x, axis=1, keepdims=True))
    logits = raw / jnp.maximum(row_n, 1e-12)

    pre = logits - thr_ref[...]
    gated = jnp.maximum(pre, 0.0)
    active = gated > 0.0

    neg = -jnp.finfo(jnp.float32).max
    masked = jnp.where(active, gated, neg)
    m = jnp.max(masked, axis=1, keepdims=True)
    e = jnp.exp(masked - m)
    probs = e / jnp.sum(e, axis=1, keepdims=True)

    probs_ref[...] = probs
    pre_ref[...] = pre
    mask_ref[...] = active.astype(jnp.float32)


@functools.partial(jax.jit)
def kernel(x, sim_matrix, gates):
    n_tokens, hidden = x.shape
    n_experts = sim_matrix.shape[1]
    gates2d = gates.reshape(1, n_experts)

    grid = (n_tokens // _BLOCK,)
    out_shape = jax.ShapeDtypeStruct((n_tokens, n_experts), jnp.float32)
    out_spec = pl.BlockSpec((_BLOCK, n_experts), lambda i: (i, 0))

    probs, pre, mask = pl.pallas_call(
        _gate_kernel,
        grid=grid,
        in_specs=[
            pl.BlockSpec((_BLOCK, hidden), lambda i: (i, 0)),
            pl.BlockSpec((hidden, n_experts), lambda i: (0, 0)),
            pl.BlockSpec((1, n_experts), lambda i: (0, 0)),
        ],
        out_specs=[out_spec, out_spec, out_spec],
        out_shape=[out_shape, out_shape, out_shape],
        scratch_shapes=[
            pltpu.VMEM((hidden, n_experts), jnp.float32),
            pltpu.VMEM((1, n_experts), jnp.float32),
        ],
        compiler_params=pltpu.CompilerParams(
            dimension_semantics=("arbitrary",),
        ),
    )(x, sim_matrix, gates2d)

    return probs, pre, mask
